# baseline (speedup 1.0000x reference)
import functools

import jax
import jax.numpy as jnp
from jax import lax
from jax.experimental import pallas as pl
from jax.experimental.pallas import tpu as pltpu
from jax.experimental.pallas import tpu_sc as plsc

B, N, S, D = 4, 32, 64, 128
T = 6 + (N - 6) * 36
NC, NS = 2, 16
NW = NC * NS
NI = N - 6
ITEMS = B * NI
MAXPEND = 16
HOT = B * 6


def _body(x_hbm, out_hbm, shared, sem):
    cid = lax.axis_index("c")
    sid = lax.axis_index("s")
    wid = sid * NC + cid

    @pl.when(sid < 12)
    def _():
        bi = sid // 3
        p = (sid % 3) * 2
        pltpu.sync_copy(
            x_hbm.at[pl.ds(bi * N + 26 + p, 2)],
            shared.at[pl.ds(bi * 6 + p, 2)],
        )
    plsc.subcore_barrier()

    pending = []

    def fire(src, dst):
        while len(pending) >= MAXPEND:
            pending.pop(0).wait()
        pending.append(pltpu.async_copy(src, dst, sem))

    @pl.when(wid < B)
    def _():
        pltpu.sync_copy(
            shared.at[pl.ds(wid * 6, 6)],
            out_hbm.at[pl.ds(wid * T, 6)],
        )

    for g in range(6):
        j0 = (wid + 8 * g) % NW

        def do_item(item, copy):
            bi = item // NI
            i = item % NI
            dst0 = bi * T + 6 + 36 * i + 6 * g
            if g > 0:
                copy(shared.at[pl.ds(bi * 6, g)],
                     out_hbm.at[pl.ds(dst0, g)])
            copy(x_hbm.at[pl.ds(bi * N + i, 1)],
                 out_hbm.at[pl.ds(dst0 + g, 1)])
            if g < 5:
                copy(shared.at[pl.ds(bi * 6 + g + 1, 5 - g)],
                     out_hbm.at[pl.ds(dst0 + g + 1, 5 - g)])

        for k in range(3):
            do_item(j0 + NW * k, fire)

        @pl.when(j0 < ITEMS - 3 * NW)
        def _():
            local = []
            do_item(j0 + NW * 3,
                    lambda s_, d_: local.append(pltpu.async_copy(s_, d_, sem)))
            for d_ in local:
                d_.wait()

    for d in pending:
        d.wait()


@functools.partial(
    pl.kernel,
    out_type=jax.ShapeDtypeStruct((B * T, S, D), jnp.float32),
    mesh=plsc.VectorSubcoreMesh(core_axis_name="c", subcore_axis_name="s"),
    scratch_types=[
        pltpu.VMEM_SHARED((HOT, S, D), jnp.float32),
        pltpu.SemaphoreType.DMA,
    ],
)
def _gather_rows(x_hbm, out_hbm, shared, sem):
    _body(x_hbm, out_hbm, shared, sem)


def kernel(x):
    b, n, s, d = x.shape
    out = _gather_rows(x.reshape(b * n, s, d))
    return out.reshape(b, 6, T // 6, s, d)


# speedup vs baseline: 9.2444x; 9.2444x over previous
import functools

import jax
import jax.numpy as jnp
from jax import lax
from jax.experimental import pallas as pl
from jax.experimental.pallas import tpu as pltpu
from jax.experimental.pallas import tpu_sc as plsc

B, N, S, D = 4, 32, 64, 128
T = 6 + (N - 6) * 36
NC, NS = 2, 16
NW = NC * NS
NI = N - 6
WPB = NW // B
MAXPEND = 16


def _body(x_hbm, out_hbm, shared, hot, sem):
    cid = lax.axis_index("c")
    sid = lax.axis_index("s")
    wid = sid * NC + cid
    bi = wid // WPB
    r = wid % WPB

    pltpu.sync_copy(x_hbm.at[pl.ds(bi * N + 26, 6)], hot)

    @pl.when(sid < 8)
    def _():
        sb = sid // 2
        off = (sid % 2) * 13
        pltpu.sync_copy(
            x_hbm.at[pl.ds(sb * N + off, 13)],
            shared.at[pl.ds(sb * NI + off, 13)],
        )
    plsc.subcore_barrier()

    pending = []

    def fire(src, dst):
        while len(pending) >= MAXPEND:
            pending.pop(0).wait()
        pending.append(pltpu.async_copy(src, dst, sem))

    @pl.when(r == 0)
    def _():
        pltpu.sync_copy(hot, out_hbm.at[pl.ds(bi * T, 6)])

    for g in range(6):
        j0 = (r + 2 * g) % WPB

        def do_item(i, copy):
            dst0 = bi * T + 6 + 36 * i + 6 * g
            if g > 0:
                copy(hot.at[pl.ds(0, g)],
                     out_hbm.at[pl.ds(dst0, g)])
            copy(shared.at[pl.ds(bi * NI + i, 1)],
                 out_hbm.at[pl.ds(dst0 + g, 1)])
            if g < 5:
                copy(hot.at[pl.ds(g + 1, 5 - g)],
                     out_hbm.at[pl.ds(dst0 + g + 1, 5 - g)])

        for k in range(3):
            do_item(j0 + WPB * k, fire)

        @pl.when(j0 < NI - 3 * WPB)
        def _():
            local = []
            do_item(j0 + WPB * 3,
                    lambda s_, d_: local.append(pltpu.async_copy(s_, d_, sem)))
            for d_ in local:
                d_.wait()

    for d in pending:
        d.wait()


@functools.partial(
    pl.kernel,
    out_type=jax.ShapeDtypeStruct((B * T, S, D), jnp.float32),
    mesh=plsc.VectorSubcoreMesh(core_axis_name="c", subcore_axis_name="s"),
    scratch_types=[
        pltpu.VMEM_SHARED((B * NI, S, D), jnp.float32),
        pltpu.VMEM((6, S, D), jnp.float32),
        pltpu.SemaphoreType.DMA,
    ],
)
def _gather_rows(x_hbm, out_hbm, shared, hot, sem):
    _body(x_hbm, out_hbm, shared, hot, sem)


def kernel(x):
    b, n, s, d = x.shape
    out = _gather_rows(x.reshape(b * n, s, d))
    return out.reshape(b, 6, T // 6, s, d)


# speedup vs baseline: 9.2499x; 1.0006x over previous
import functools

import jax
import jax.numpy as jnp
from jax import lax
from jax.experimental import pallas as pl
from jax.experimental.pallas import tpu as pltpu
from jax.experimental.pallas import tpu_sc as plsc

B, N, S, D = 4, 32, 64, 128
T = 6 + (N - 6) * 36
NC, NS = 2, 16
NW = NC * NS
NI = N - 6
WPB = NW // B
MAXPEND = 16


def _body(x_hbm, out_hbm, hot, cold, sem):
    cid = lax.axis_index("c")
    sid = lax.axis_index("s")
    wid = sid * NC + cid
    bi = wid // WPB
    r = wid % WPB
    ra = r
    rb = (r + 4) % WPB

    stage = []

    stage.append(pltpu.async_copy(
        x_hbm.at[pl.ds(bi * N + 26, 6)], hot, sem))

    for base, slot0 in ((ra, 0), (rb, 4)):
        for k in range(3):
            stage.append(pltpu.async_copy(
                x_hbm.at[pl.ds(bi * N + base + 8 * k, 1)],
                cold.at[pl.ds(slot0 + k, 1)], sem))

        @pl.when(base < NI - 3 * WPB)
        def _(base=base, slot0=slot0):
            pltpu.sync_copy(
                x_hbm.at[pl.ds(bi * N + base + 24, 1)],
                cold.at[pl.ds(slot0 + 3, 1)])
    for d in stage:
        d.wait()

    pending = []

    def fire(src, dst):
        while len(pending) >= MAXPEND:
            pending.pop(0).wait()
        pending.append(pltpu.async_copy(src, dst, sem))

    @pl.when(r == 0)
    def _():
        pltpu.sync_copy(hot, out_hbm.at[pl.ds(bi * T, 6)])

    for g in range(6):
        j0 = ra if g % 2 == 0 else rb
        slot0 = 0 if g % 2 == 0 else 4

        def do_item(i, k, copy):
            dst0 = bi * T + 6 + 36 * i + 6 * g
            if g > 0:
                copy(hot.at[pl.ds(0, g)],
                     out_hbm.at[pl.ds(dst0, g)])
            copy(cold.at[pl.ds(slot0 + k, 1)],
                 out_hbm.at[pl.ds(dst0 + g, 1)])
            if g < 5:
                copy(hot.at[pl.ds(g + 1, 5 - g)],
                     out_hbm.at[pl.ds(dst0 + g + 1, 5 - g)])

        for k in range(3):
            do_item(j0 + WPB * k, k, fire)

        @pl.when(j0 < NI - 3 * WPB)
        def _():
            local = []
            do_item(j0 + WPB * 3, 3,
                    lambda s_, d_: local.append(pltpu.async_copy(s_, d_, sem)))
            for d_ in local:
                d_.wait()

    for d in pending:
        d.wait()


@functools.partial(
    pl.kernel,
    out_type=jax.ShapeDtypeStruct((B * T, S, D), jnp.float32),
    mesh=plsc.VectorSubcoreMesh(core_axis_name="c", subcore_axis_name="s"),
    scratch_types=[
        pltpu.VMEM((6, S, D), jnp.float32),
        pltpu.VMEM((8, S, D), jnp.float32),
        pltpu.SemaphoreType.DMA,
    ],
)
def _gather_rows(x_hbm, out_hbm, hot, cold, sem):
    _body(x_hbm, out_hbm, hot, cold, sem)


def kernel(x):
    b, n, s, d = x.shape
    out = _gather_rows(x.reshape(b * n, s, d))
    return out.reshape(b, 6, T // 6, s, d)
